# baseline (speedup 1.0000x reference)
import functools

import jax
import jax.numpy as jnp
from jax.experimental import pallas as pl
from jax.experimental.pallas import tpu as pltpu

_VMEM_LIMIT = 64 * 1024 * 1024


def _stats_kernel(x_ref, psum_ref, psq_ref, *, n_rows, tile, steps_per_split,
                  needs_mask):
    k = pl.program_id(1)

    @pl.when(k == 0)
    def _():
        psum_ref[...] = jnp.zeros_like(psum_ref)
        psq_ref[...] = jnp.zeros_like(psq_ref)

    x = x_ref[...].astype(jnp.float32)
    if needs_mask:
        j = pl.program_id(0)
        row0 = (j * steps_per_split + k) * tile
        rows = row0 + jax.lax.broadcasted_iota(jnp.int32, (tile, 1), 0)
        x = jnp.where(rows < n_rows, x, 0.0)

    xr = x.reshape(tile // 8, 8, x.shape[-1])
    psum_ref[...] += jnp.sum(xr, axis=0)
    psq_ref[...] += jnp.sum(xr * xr, axis=0)


def _apply_kernel(x_ref, psum_ref, psq_ref, gamma_ref, beta_ref, o_ref, *,
                  inv_n, eps):
    mean = jnp.sum(psum_ref[...], axis=0, keepdims=True) * inv_n
    msq = jnp.sum(psq_ref[...], axis=0, keepdims=True) * inv_n
    var = jnp.maximum(msq - mean * mean, 0.0)
    scale = gamma_ref[...] * jax.lax.rsqrt(var + eps)
    shift = beta_ref[...] - mean * scale
    o_ref[...] = (x_ref[...].astype(jnp.float32) * scale
                  + shift).astype(o_ref.dtype)


def _round_up(a, m):
    return ((a + m - 1) // m) * m


def _fused_kernel(x_ref, gamma_ref, beta_ref, o_ref, xs_ref, psum_ref,
                  psq_ref, ss_ref, *, s, tile, hc, inv_n, eps):
    k = pl.program_id(1)

    @pl.when(k == 0)
    def _():
        psum_ref[...] = jnp.zeros_like(psum_ref)
        psq_ref[...] = jnp.zeros_like(psq_ref)

    @pl.when(k < s)
    def _():
        x = x_ref[...]
        xr = x.reshape(tile // 8, 8, hc)
        psum_ref[...] += jnp.sum(xr, axis=0)
        psq_ref[...] += jnp.sum(xr * xr, axis=0)

        @pl.when(k < s - 1)
        def _():
            kk = jnp.minimum(k, s - 2)
            xs_ref[pl.ds(kk * tile, tile), :] = x

        @pl.when(k == s - 1)
        def _():
            mean = jnp.sum(psum_ref[...], axis=0, keepdims=True) * inv_n
            msq = jnp.sum(psq_ref[...], axis=0, keepdims=True) * inv_n
            var = jnp.maximum(msq - mean * mean, 0.0)
            scale = gamma_ref[...] * jax.lax.rsqrt(var + eps)
            shift = beta_ref[...] - mean * scale
            ss_ref[0:1, :] = scale
            ss_ref[1:2, :] = shift
            o_ref[...] = x * scale + shift

    @pl.when(k >= s)
    def _():
        r = jnp.maximum(k - s, 0)
        scale = ss_ref[0:1, :]
        shift = ss_ref[1:2, :]
        xb = xs_ref[pl.ds(r * tile, tile), :]
        o_ref[...] = xb * scale + shift


def _fused_single_read(x2, gamma2, beta2, *, tile, eps):
    R, H = x2.shape
    hc = H // 2
    s = R // tile
    inv_n = 1.0 / R

    x_spec = pl.BlockSpec((tile, hc), lambda j, k: (jnp.minimum(k, s - 1), j))
    o_spec = pl.BlockSpec((tile, hc),
                          lambda j, k: (jnp.where(k >= s, k - s, s - 1), j))
    chan_spec = pl.BlockSpec((1, hc), lambda j, k: (0, j))
    return pl.pallas_call(
        functools.partial(_fused_kernel, s=s, tile=tile, hc=hc,
                          inv_n=inv_n, eps=eps),
        grid=(2, 2 * s - 1),
        in_specs=[x_spec, chan_spec, chan_spec],
        out_specs=o_spec,
        out_shape=jax.ShapeDtypeStruct((R, H), x2.dtype),
        scratch_shapes=[
            pltpu.VMEM(((s - 1) * tile, hc), jnp.float32),
            pltpu.VMEM((8, hc), jnp.float32),
            pltpu.VMEM((8, hc), jnp.float32),
            pltpu.VMEM((2, hc), jnp.float32),
        ],
        compiler_params=pltpu.CompilerParams(
            dimension_semantics=("parallel", "arbitrary"),
            vmem_limit_bytes=_VMEM_LIMIT),
    )(x2, gamma2, beta2)


def _manual_kernel(x_hbm, gamma_ref, beta_ref, o_hbm, xs_ref, in_sems,
                   out_sems, *, s, tile, hc, nc, inv_n, eps):
    j = pl.program_id(0)
    hw = hc // nc

    for c in range(nc):
        col = j * hc + c * hw
        for i in range(s):
            pltpu.make_async_copy(
                x_hbm.at[pl.ds(i * tile, tile), pl.ds(col, hw)],
                xs_ref.at[c, i], in_sems.at[c, i]).start()

    for c in range(nc):
        col = j * hc + c * hw
        acc_s = jnp.zeros((8, hw), jnp.float32)
        acc_q = jnp.zeros((8, hw), jnp.float32)
        for i in range(s):
            pltpu.make_async_copy(xs_ref.at[c, i], xs_ref.at[c, i],
                                  in_sems.at[c, i]).wait()
            xr = xs_ref[c, i].reshape(tile // 8, 8, hw)
            acc_s = acc_s + jnp.sum(xr, axis=0)
            acc_q = acc_q + jnp.sum(xr * xr, axis=0)
        mean = jnp.sum(acc_s, axis=0, keepdims=True) * inv_n
        msq = jnp.sum(acc_q, axis=0, keepdims=True) * inv_n
        var = jnp.maximum(msq - mean * mean, 0.0)
        scale = gamma_ref[0:1, c * hw:(c + 1) * hw] * jax.lax.rsqrt(var + eps)
        shift = beta_ref[0:1, c * hw:(c + 1) * hw] - mean * scale
        for i in range(s):
            xs_ref[c, i] = xs_ref[c, i] * scale + shift
            pltpu.make_async_copy(
                xs_ref.at[c, i],
                o_hbm.at[pl.ds(i * tile, tile), pl.ds(col, hw)],
                out_sems.at[c, i]).start()

    for c in range(nc):
        for i in range(s):
            pltpu.make_async_copy(xs_ref.at[c, i], xs_ref.at[c, i],
                                  out_sems.at[c, i]).wait()


def _manual_single_read(x2, gamma2, beta2, *, tile, nc, eps):
    R, H = x2.shape
    hc = H // 2
    hw = hc // nc
    s = R // tile
    inv_n = 1.0 / R

    chan_spec = pl.BlockSpec((1, hc), lambda j: (0, j))
    return pl.pallas_call(
        functools.partial(_manual_kernel, s=s, tile=tile, hc=hc, nc=nc,
                          inv_n=inv_n, eps=eps),
        grid=(2,),
        in_specs=[pl.BlockSpec(memory_space=pl.ANY), chan_spec, chan_spec],
        out_specs=pl.BlockSpec(memory_space=pl.ANY),
        out_shape=jax.ShapeDtypeStruct((R, H), x2.dtype),
        scratch_shapes=[
            pltpu.VMEM((nc, s, tile, hw), jnp.float32),
            pltpu.SemaphoreType.DMA((nc, s)),
            pltpu.SemaphoreType.DMA((nc, s)),
        ],
        compiler_params=pltpu.CompilerParams(
            dimension_semantics=("parallel",),
            vmem_limit_bytes=_VMEM_LIMIT),
    )(x2, gamma2, beta2)


def kernel(x, gamma, beta, *, eps=1e-5):
    orig_shape = x.shape
    H = orig_shape[-1]
    x2 = x.reshape(-1, H)
    R = x2.shape[0]
    itemsize = jnp.dtype(x.dtype).itemsize

    gamma2 = gamma.reshape(1, H).astype(jnp.float32)
    beta2 = beta.reshape(1, H).astype(jnp.float32)

    if itemsize == 4 and H % 256 == 0:
        hc = H // 2
        ftile = max(8, (8 * 1024 * 1024 // (hc * 4)) // 8 * 8)
        if (R % ftile == 0 and R // ftile >= 2 and hc % 256 == 0
                and R * hc * 4 <= 36 * 1024 * 1024):
            y2 = _manual_single_read(x2.astype(jnp.float32), gamma2, beta2,
                                     tile=ftile, nc=1, eps=eps)
            return y2.reshape(orig_shape).astype(x.dtype)

    align = 8 if itemsize == 4 else (16 if itemsize == 2 else 32)
    target_bytes = 8 * 1024 * 1024
    bytes_per_row = H * itemsize
    tile = max(align, (target_bytes // bytes_per_row) // align * align)
    tile = min(tile, _round_up(R, align))

    steps_total = pl.cdiv(R, tile)
    splits = 2 if steps_total >= 2 else 1
    steps_per_split = pl.cdiv(steps_total, splits)
    covered = splits * steps_per_split
    needs_mask = covered * tile != R
    needs_clamp = covered > steps_total
    last_block = steps_total - 1

    if needs_clamp:
        def x_stats_map(j, k):
            return (jnp.minimum(j * steps_per_split + k, last_block), 0)
    else:
        def x_stats_map(j, k):
            return (j * steps_per_split + k, 0)

    psum, psq = pl.pallas_call(
        functools.partial(_stats_kernel, n_rows=R, tile=tile,
                          steps_per_split=steps_per_split,
                          needs_mask=needs_mask),
        grid=(splits, steps_per_split),
        in_specs=[pl.BlockSpec((tile, H), x_stats_map)],
        out_specs=(pl.BlockSpec((None, 8, H), lambda j, k: (j, 0, 0)),
                   pl.BlockSpec((None, 8, H), lambda j, k: (j, 0, 0))),
        out_shape=(jax.ShapeDtypeStruct((splits, 8, H), jnp.float32),
                   jax.ShapeDtypeStruct((splits, 8, H), jnp.float32)),
        compiler_params=pltpu.CompilerParams(
            dimension_semantics=("parallel", "arbitrary"),
            vmem_limit_bytes=_VMEM_LIMIT),
    )(x2)

    psum2 = psum.reshape(splits * 8, H)
    psq2 = psq.reshape(splits * 8, H)
    atile = max(align, (8 * 1024 * 1024 // bytes_per_row) // align * align)
    atile = min(atile, _round_up(R, align))
    asteps = pl.cdiv(R, atile)
    row_spec = pl.BlockSpec((atile, H), lambda i: (i, 0))
    part_spec = pl.BlockSpec((splits * 8, H), lambda i: (0, 0))
    chan_spec = pl.BlockSpec((1, H), lambda i: (0, 0))
    y2 = pl.pallas_call(
        functools.partial(_apply_kernel, inv_n=1.0 / R, eps=eps),
        grid=(asteps,),
        in_specs=[row_spec, part_spec, part_spec, chan_spec, chan_spec],
        out_specs=row_spec,
        out_shape=jax.ShapeDtypeStruct((R, H), x.dtype),
        compiler_params=pltpu.CompilerParams(
            dimension_semantics=("parallel",),
            vmem_limit_bytes=_VMEM_LIMIT),
    )(x2, psum2, psq2, gamma2, beta2)

    return y2.reshape(orig_shape)


# speedup vs baseline: 1.1299x; 1.1299x over previous
import functools

import jax
import jax.numpy as jnp
from jax.experimental import pallas as pl
from jax.experimental.pallas import tpu as pltpu

_VMEM_LIMIT = 64 * 1024 * 1024


def _stats_kernel(x_ref, psum_ref, psq_ref, *, n_rows, tile, steps_per_split,
                  needs_mask):
    k = pl.program_id(1)

    @pl.when(k == 0)
    def _():
        psum_ref[...] = jnp.zeros_like(psum_ref)
        psq_ref[...] = jnp.zeros_like(psq_ref)

    x = x_ref[...].astype(jnp.float32)
    if needs_mask:
        j = pl.program_id(0)
        row0 = (j * steps_per_split + k) * tile
        rows = row0 + jax.lax.broadcasted_iota(jnp.int32, (tile, 1), 0)
        x = jnp.where(rows < n_rows, x, 0.0)

    xr = x.reshape(tile // 8, 8, x.shape[-1])
    psum_ref[...] += jnp.sum(xr, axis=0)
    psq_ref[...] += jnp.sum(xr * xr, axis=0)


def _apply_kernel(x_ref, psum_ref, psq_ref, gamma_ref, beta_ref, o_ref, *,
                  inv_n, eps):
    mean = jnp.sum(psum_ref[...], axis=0, keepdims=True) * inv_n
    msq = jnp.sum(psq_ref[...], axis=0, keepdims=True) * inv_n
    var = jnp.maximum(msq - mean * mean, 0.0)
    scale = gamma_ref[...] * jax.lax.rsqrt(var + eps)
    shift = beta_ref[...] - mean * scale
    o_ref[...] = (x_ref[...].astype(jnp.float32) * scale
                  + shift).astype(o_ref.dtype)


def _round_up(a, m):
    return ((a + m - 1) // m) * m


def _fused_kernel(x_ref, gamma_ref, beta_ref, o_ref, xs_ref, psum_ref,
                  psq_ref, ss_ref, *, s, tile, hc, inv_n, eps):
    k = pl.program_id(1)

    @pl.when(k == 0)
    def _():
        psum_ref[...] = jnp.zeros_like(psum_ref)
        psq_ref[...] = jnp.zeros_like(psq_ref)

    @pl.when(k < s)
    def _():
        x = x_ref[...]
        xr = x.reshape(tile // 8, 8, hc)
        psum_ref[...] += jnp.sum(xr, axis=0)
        psq_ref[...] += jnp.sum(xr * xr, axis=0)

        @pl.when(k < s - 1)
        def _():
            kk = jnp.minimum(k, s - 2)
            xs_ref[pl.ds(kk * tile, tile), :] = x

        @pl.when(k == s - 1)
        def _():
            mean = jnp.sum(psum_ref[...], axis=0, keepdims=True) * inv_n
            msq = jnp.sum(psq_ref[...], axis=0, keepdims=True) * inv_n
            var = jnp.maximum(msq - mean * mean, 0.0)
            scale = gamma_ref[...] * jax.lax.rsqrt(var + eps)
            shift = beta_ref[...] - mean * scale
            ss_ref[0:1, :] = scale
            ss_ref[1:2, :] = shift
            o_ref[...] = x * scale + shift

    @pl.when(k >= s)
    def _():
        r = jnp.maximum(k - s, 0)
        scale = ss_ref[0:1, :]
        shift = ss_ref[1:2, :]
        xb = xs_ref[pl.ds(r * tile, tile), :]
        o_ref[...] = xb * scale + shift


def _fused_single_read(x2, gamma2, beta2, *, tile, eps):
    R, H = x2.shape
    hc = H // 2
    s = R // tile
    inv_n = 1.0 / R

    x_spec = pl.BlockSpec((tile, hc), lambda j, k: (jnp.minimum(k, s - 1), j))
    o_spec = pl.BlockSpec((tile, hc),
                          lambda j, k: (jnp.where(k >= s, k - s, s - 1), j))
    chan_spec = pl.BlockSpec((1, hc), lambda j, k: (0, j))
    return pl.pallas_call(
        functools.partial(_fused_kernel, s=s, tile=tile, hc=hc,
                          inv_n=inv_n, eps=eps),
        grid=(2, 2 * s - 1),
        in_specs=[x_spec, chan_spec, chan_spec],
        out_specs=o_spec,
        out_shape=jax.ShapeDtypeStruct((R, H), x2.dtype),
        scratch_shapes=[
            pltpu.VMEM(((s - 1) * tile, hc), jnp.float32),
            pltpu.VMEM((8, hc), jnp.float32),
            pltpu.VMEM((8, hc), jnp.float32),
            pltpu.VMEM((2, hc), jnp.float32),
        ],
        compiler_params=pltpu.CompilerParams(
            dimension_semantics=("parallel", "arbitrary"),
            vmem_limit_bytes=_VMEM_LIMIT),
    )(x2, gamma2, beta2)


def _manual_kernel(x_hbm, gamma_ref, beta_ref, o_hbm, xs_ref, in_sems,
                   out_sems, *, s, tile, hc, nc, inv_n, eps):
    j = pl.program_id(0)
    hw = hc // nc

    for c in range(nc):
        col = j * hc + c * hw
        for i in range(s):
            pltpu.make_async_copy(
                x_hbm.at[pl.ds(i * tile, tile), pl.ds(col, hw)],
                xs_ref.at[c, i], in_sems.at[c, i]).start()

    for c in range(nc):
        col = j * hc + c * hw
        acc_s = jnp.zeros((8, hw), jnp.float32)
        acc_q = jnp.zeros((8, hw), jnp.float32)
        for i in range(s):
            pltpu.make_async_copy(xs_ref.at[c, i], xs_ref.at[c, i],
                                  in_sems.at[c, i]).wait()
            xr = xs_ref[c, i].reshape(tile // 8, 8, hw)
            acc_s = acc_s + jnp.sum(xr, axis=0)
            acc_q = acc_q + jnp.sum(xr * xr, axis=0)
        mean = jnp.sum(acc_s, axis=0, keepdims=True) * inv_n
        msq = jnp.sum(acc_q, axis=0, keepdims=True) * inv_n
        var = jnp.maximum(msq - mean * mean, 0.0)
        scale = gamma_ref[0:1, c * hw:(c + 1) * hw] * jax.lax.rsqrt(var + eps)
        shift = beta_ref[0:1, c * hw:(c + 1) * hw] - mean * scale
        for i in range(s):
            xs_ref[c, i] = xs_ref[c, i] * scale + shift
            pltpu.make_async_copy(
                xs_ref.at[c, i],
                o_hbm.at[pl.ds(i * tile, tile), pl.ds(col, hw)],
                out_sems.at[c, i]).start()

    for c in range(nc):
        for i in range(s):
            pltpu.make_async_copy(xs_ref.at[c, i], xs_ref.at[c, i],
                                  out_sems.at[c, i]).wait()


def _manual_single_read(x2, gamma2, beta2, *, tile, nc, eps):
    R, H = x2.shape
    hc = H // 2
    hw = hc // nc
    s = R // tile
    inv_n = 1.0 / R

    chan_spec = pl.BlockSpec((1, hc), lambda j: (0, j))
    return pl.pallas_call(
        functools.partial(_manual_kernel, s=s, tile=tile, hc=hc, nc=nc,
                          inv_n=inv_n, eps=eps),
        grid=(2,),
        in_specs=[pl.BlockSpec(memory_space=pl.ANY), chan_spec, chan_spec],
        out_specs=pl.BlockSpec(memory_space=pl.ANY),
        out_shape=jax.ShapeDtypeStruct((R, H), x2.dtype),
        scratch_shapes=[
            pltpu.VMEM((nc, s, tile, hw), jnp.float32),
            pltpu.SemaphoreType.DMA((nc, s)),
            pltpu.SemaphoreType.DMA((nc, s)),
        ],
        compiler_params=pltpu.CompilerParams(
            dimension_semantics=("parallel",),
            vmem_limit_bytes=_VMEM_LIMIT),
    )(x2, gamma2, beta2)


def kernel(x, gamma, beta, *, eps=1e-5):
    orig_shape = x.shape
    H = orig_shape[-1]
    x2 = x.reshape(-1, H)
    R = x2.shape[0]
    itemsize = jnp.dtype(x.dtype).itemsize

    gamma2 = gamma.reshape(1, H).astype(jnp.float32)
    beta2 = beta.reshape(1, H).astype(jnp.float32)

    if itemsize == 4 and H % 256 == 0:
        hc = H // 2
        ftile = max(8, (4 * 1024 * 1024 // (hc * 4)) // 8 * 8)
        if (R % ftile == 0 and R // ftile >= 2 and hc % 256 == 0
                and R * hc * 4 <= 36 * 1024 * 1024):
            y2 = _manual_single_read(x2.astype(jnp.float32), gamma2, beta2,
                                     tile=ftile, nc=2, eps=eps)
            return y2.reshape(orig_shape).astype(x.dtype)

    align = 8 if itemsize == 4 else (16 if itemsize == 2 else 32)
    target_bytes = 8 * 1024 * 1024
    bytes_per_row = H * itemsize
    tile = max(align, (target_bytes // bytes_per_row) // align * align)
    tile = min(tile, _round_up(R, align))

    steps_total = pl.cdiv(R, tile)
    splits = 2 if steps_total >= 2 else 1
    steps_per_split = pl.cdiv(steps_total, splits)
    covered = splits * steps_per_split
    needs_mask = covered * tile != R
    needs_clamp = covered > steps_total
    last_block = steps_total - 1

    if needs_clamp:
        def x_stats_map(j, k):
            return (jnp.minimum(j * steps_per_split + k, last_block), 0)
    else:
        def x_stats_map(j, k):
            return (j * steps_per_split + k, 0)

    psum, psq = pl.pallas_call(
        functools.partial(_stats_kernel, n_rows=R, tile=tile,
                          steps_per_split=steps_per_split,
                          needs_mask=needs_mask),
        grid=(splits, steps_per_split),
        in_specs=[pl.BlockSpec((tile, H), x_stats_map)],
        out_specs=(pl.BlockSpec((None, 8, H), lambda j, k: (j, 0, 0)),
                   pl.BlockSpec((None, 8, H), lambda j, k: (j, 0, 0))),
        out_shape=(jax.ShapeDtypeStruct((splits, 8, H), jnp.float32),
                   jax.ShapeDtypeStruct((splits, 8, H), jnp.float32)),
        compiler_params=pltpu.CompilerParams(
            dimension_semantics=("parallel", "arbitrary"),
            vmem_limit_bytes=_VMEM_LIMIT),
    )(x2)

    psum2 = psum.reshape(splits * 8, H)
    psq2 = psq.reshape(splits * 8, H)
    atile = max(align, (8 * 1024 * 1024 // bytes_per_row) // align * align)
    atile = min(atile, _round_up(R, align))
    asteps = pl.cdiv(R, atile)
    row_spec = pl.BlockSpec((atile, H), lambda i: (i, 0))
    part_spec = pl.BlockSpec((splits * 8, H), lambda i: (0, 0))
    chan_spec = pl.BlockSpec((1, H), lambda i: (0, 0))
    y2 = pl.pallas_call(
        functools.partial(_apply_kernel, inv_n=1.0 / R, eps=eps),
        grid=(asteps,),
        in_specs=[row_spec, part_spec, part_spec, chan_spec, chan_spec],
        out_specs=row_spec,
        out_shape=jax.ShapeDtypeStruct((R, H), x.dtype),
        compiler_params=pltpu.CompilerParams(
            dimension_semantics=("parallel",),
            vmem_limit_bytes=_VMEM_LIMIT),
    )(x2, psum2, psq2, gamma2, beta2)

    return y2.reshape(orig_shape)


# speedup vs baseline: 1.1427x; 1.0113x over previous
import functools

import jax
import jax.numpy as jnp
from jax.experimental import pallas as pl
from jax.experimental.pallas import tpu as pltpu

_VMEM_LIMIT = 64 * 1024 * 1024


def _stats_kernel(x_ref, psum_ref, psq_ref, *, n_rows, tile, steps_per_split,
                  needs_mask):
    k = pl.program_id(1)

    @pl.when(k == 0)
    def _():
        psum_ref[...] = jnp.zeros_like(psum_ref)
        psq_ref[...] = jnp.zeros_like(psq_ref)

    x = x_ref[...].astype(jnp.float32)
    if needs_mask:
        j = pl.program_id(0)
        row0 = (j * steps_per_split + k) * tile
        rows = row0 + jax.lax.broadcasted_iota(jnp.int32, (tile, 1), 0)
        x = jnp.where(rows < n_rows, x, 0.0)

    xr = x.reshape(tile // 8, 8, x.shape[-1])
    psum_ref[...] += jnp.sum(xr, axis=0)
    psq_ref[...] += jnp.sum(xr * xr, axis=0)


def _apply_kernel(x_ref, psum_ref, psq_ref, gamma_ref, beta_ref, o_ref, *,
                  inv_n, eps):
    mean = jnp.sum(psum_ref[...], axis=0, keepdims=True) * inv_n
    msq = jnp.sum(psq_ref[...], axis=0, keepdims=True) * inv_n
    var = jnp.maximum(msq - mean * mean, 0.0)
    scale = gamma_ref[...] * jax.lax.rsqrt(var + eps)
    shift = beta_ref[...] - mean * scale
    o_ref[...] = (x_ref[...].astype(jnp.float32) * scale
                  + shift).astype(o_ref.dtype)


def _round_up(a, m):
    return ((a + m - 1) // m) * m


def _fused_kernel(x_ref, gamma_ref, beta_ref, o_ref, xs_ref, psum_ref,
                  psq_ref, ss_ref, *, s, tile, hc, inv_n, eps):
    k = pl.program_id(1)

    @pl.when(k == 0)
    def _():
        psum_ref[...] = jnp.zeros_like(psum_ref)
        psq_ref[...] = jnp.zeros_like(psq_ref)

    @pl.when(k < s)
    def _():
        x = x_ref[...]
        xr = x.reshape(tile // 8, 8, hc)
        psum_ref[...] += jnp.sum(xr, axis=0)
        psq_ref[...] += jnp.sum(xr * xr, axis=0)

        @pl.when(k < s - 1)
        def _():
            kk = jnp.minimum(k, s - 2)
            xs_ref[pl.ds(kk * tile, tile), :] = x

        @pl.when(k == s - 1)
        def _():
            mean = jnp.sum(psum_ref[...], axis=0, keepdims=True) * inv_n
            msq = jnp.sum(psq_ref[...], axis=0, keepdims=True) * inv_n
            var = jnp.maximum(msq - mean * mean, 0.0)
            scale = gamma_ref[...] * jax.lax.rsqrt(var + eps)
            shift = beta_ref[...] - mean * scale
            ss_ref[0:1, :] = scale
            ss_ref[1:2, :] = shift
            o_ref[...] = x * scale + shift

    @pl.when(k >= s)
    def _():
        r = jnp.maximum(k - s, 0)
        scale = ss_ref[0:1, :]
        shift = ss_ref[1:2, :]
        xb = xs_ref[pl.ds(r * tile, tile), :]
        o_ref[...] = xb * scale + shift


def _fused_single_read(x2, gamma2, beta2, *, tile, eps):
    R, H = x2.shape
    hc = H // 2
    s = R // tile
    inv_n = 1.0 / R

    x_spec = pl.BlockSpec((tile, hc), lambda j, k: (jnp.minimum(k, s - 1), j))
    o_spec = pl.BlockSpec((tile, hc),
                          lambda j, k: (jnp.where(k >= s, k - s, s - 1), j))
    chan_spec = pl.BlockSpec((1, hc), lambda j, k: (0, j))
    return pl.pallas_call(
        functools.partial(_fused_kernel, s=s, tile=tile, hc=hc,
                          inv_n=inv_n, eps=eps),
        grid=(2, 2 * s - 1),
        in_specs=[x_spec, chan_spec, chan_spec],
        out_specs=o_spec,
        out_shape=jax.ShapeDtypeStruct((R, H), x2.dtype),
        scratch_shapes=[
            pltpu.VMEM(((s - 1) * tile, hc), jnp.float32),
            pltpu.VMEM((8, hc), jnp.float32),
            pltpu.VMEM((8, hc), jnp.float32),
            pltpu.VMEM((2, hc), jnp.float32),
        ],
        compiler_params=pltpu.CompilerParams(
            dimension_semantics=("parallel", "arbitrary"),
            vmem_limit_bytes=_VMEM_LIMIT),
    )(x2, gamma2, beta2)


def _manual_kernel(x_hbm, gamma_ref, beta_ref, o_hbm, xs_ref, in_sems,
                   out_sems, *, s, tile, hc, nc, inv_n, eps):
    j = pl.program_id(0)
    hw = hc // nc

    for c in range(nc):
        col = j * hc + c * hw
        for i in range(s):
            pltpu.make_async_copy(
                x_hbm.at[pl.ds(i * tile, tile), pl.ds(col, hw)],
                xs_ref.at[c, i], in_sems.at[c, i]).start()

    for c in range(nc):
        col = j * hc + c * hw
        acc_s = jnp.zeros((8, hw), jnp.float32)
        acc_q = jnp.zeros((8, hw), jnp.float32)
        for i in range(s):
            pltpu.make_async_copy(xs_ref.at[c, i], xs_ref.at[c, i],
                                  in_sems.at[c, i]).wait()
            xr = xs_ref[c, i].reshape(tile // 8, 8, hw)
            acc_s = acc_s + jnp.sum(xr, axis=0)
            acc_q = acc_q + jnp.sum(xr * xr, axis=0)
        mean = jnp.sum(acc_s, axis=0, keepdims=True) * inv_n
        msq = jnp.sum(acc_q, axis=0, keepdims=True) * inv_n
        var = jnp.maximum(msq - mean * mean, 0.0)
        scale = gamma_ref[0:1, c * hw:(c + 1) * hw] * jax.lax.rsqrt(var + eps)
        shift = beta_ref[0:1, c * hw:(c + 1) * hw] - mean * scale
        for i in range(s):
            xs_ref[c, i] = xs_ref[c, i] * scale + shift
            pltpu.make_async_copy(
                xs_ref.at[c, i],
                o_hbm.at[pl.ds(i * tile, tile), pl.ds(col, hw)],
                out_sems.at[c, i]).start()

    for c in range(nc):
        for i in range(s):
            pltpu.make_async_copy(xs_ref.at[c, i], xs_ref.at[c, i],
                                  out_sems.at[c, i]).wait()


def _manual_single_read(x2, gamma2, beta2, *, tile, nc, eps):
    R, H = x2.shape
    hc = H // 2
    hw = hc // nc
    s = R // tile
    inv_n = 1.0 / R

    chan_spec = pl.BlockSpec((1, hc), lambda j: (0, j))
    return pl.pallas_call(
        functools.partial(_manual_kernel, s=s, tile=tile, hc=hc, nc=nc,
                          inv_n=inv_n, eps=eps),
        grid=(2,),
        in_specs=[pl.BlockSpec(memory_space=pl.ANY), chan_spec, chan_spec],
        out_specs=pl.BlockSpec(memory_space=pl.ANY),
        out_shape=jax.ShapeDtypeStruct((R, H), x2.dtype),
        scratch_shapes=[
            pltpu.VMEM((nc, s, tile, hw), jnp.float32),
            pltpu.SemaphoreType.DMA((nc, s)),
            pltpu.SemaphoreType.DMA((nc, s)),
        ],
        compiler_params=pltpu.CompilerParams(
            dimension_semantics=("parallel",),
            vmem_limit_bytes=_VMEM_LIMIT),
    )(x2, gamma2, beta2)


def kernel(x, gamma, beta, *, eps=1e-5):
    orig_shape = x.shape
    H = orig_shape[-1]
    x2 = x.reshape(-1, H)
    R = x2.shape[0]
    itemsize = jnp.dtype(x.dtype).itemsize

    gamma2 = gamma.reshape(1, H).astype(jnp.float32)
    beta2 = beta.reshape(1, H).astype(jnp.float32)

    if itemsize == 4 and H % 256 == 0:
        hc = H // 2
        ftile = max(8, (8 * 1024 * 1024 // (hc * 4)) // 8 * 8)
        if (R % ftile == 0 and R // ftile >= 2 and hc % 256 == 0
                and R * hc * 4 <= 36 * 1024 * 1024):
            y2 = _manual_single_read(x2.astype(jnp.float32), gamma2, beta2,
                                     tile=ftile, nc=2, eps=eps)
            return y2.reshape(orig_shape).astype(x.dtype)

    align = 8 if itemsize == 4 else (16 if itemsize == 2 else 32)
    target_bytes = 8 * 1024 * 1024
    bytes_per_row = H * itemsize
    tile = max(align, (target_bytes // bytes_per_row) // align * align)
    tile = min(tile, _round_up(R, align))

    steps_total = pl.cdiv(R, tile)
    splits = 2 if steps_total >= 2 else 1
    steps_per_split = pl.cdiv(steps_total, splits)
    covered = splits * steps_per_split
    needs_mask = covered * tile != R
    needs_clamp = covered > steps_total
    last_block = steps_total - 1

    if needs_clamp:
        def x_stats_map(j, k):
            return (jnp.minimum(j * steps_per_split + k, last_block), 0)
    else:
        def x_stats_map(j, k):
            return (j * steps_per_split + k, 0)

    psum, psq = pl.pallas_call(
        functools.partial(_stats_kernel, n_rows=R, tile=tile,
                          steps_per_split=steps_per_split,
                          needs_mask=needs_mask),
        grid=(splits, steps_per_split),
        in_specs=[pl.BlockSpec((tile, H), x_stats_map)],
        out_specs=(pl.BlockSpec((None, 8, H), lambda j, k: (j, 0, 0)),
                   pl.BlockSpec((None, 8, H), lambda j, k: (j, 0, 0))),
        out_shape=(jax.ShapeDtypeStruct((splits, 8, H), jnp.float32),
                   jax.ShapeDtypeStruct((splits, 8, H), jnp.float32)),
        compiler_params=pltpu.CompilerParams(
            dimension_semantics=("parallel", "arbitrary"),
            vmem_limit_bytes=_VMEM_LIMIT),
    )(x2)

    psum2 = psum.reshape(splits * 8, H)
    psq2 = psq.reshape(splits * 8, H)
    atile = max(align, (8 * 1024 * 1024 // bytes_per_row) // align * align)
    atile = min(atile, _round_up(R, align))
    asteps = pl.cdiv(R, atile)
    row_spec = pl.BlockSpec((atile, H), lambda i: (i, 0))
    part_spec = pl.BlockSpec((splits * 8, H), lambda i: (0, 0))
    chan_spec = pl.BlockSpec((1, H), lambda i: (0, 0))
    y2 = pl.pallas_call(
        functools.partial(_apply_kernel, inv_n=1.0 / R, eps=eps),
        grid=(asteps,),
        in_specs=[row_spec, part_spec, part_spec, chan_spec, chan_spec],
        out_specs=row_spec,
        out_shape=jax.ShapeDtypeStruct((R, H), x.dtype),
        compiler_params=pltpu.CompilerParams(
            dimension_semantics=("parallel",),
            vmem_limit_bytes=_VMEM_LIMIT),
    )(x2, psum2, psq2, gamma2, beta2)

    return y2.reshape(orig_shape)


# speedup vs baseline: 1.1471x; 1.0039x over previous
import functools

import jax
import jax.numpy as jnp
from jax.experimental import pallas as pl
from jax.experimental.pallas import tpu as pltpu

_VMEM_LIMIT = 64 * 1024 * 1024


def _stats_kernel(x_ref, psum_ref, psq_ref, *, n_rows, tile, steps_per_split,
                  needs_mask):
    k = pl.program_id(1)

    @pl.when(k == 0)
    def _():
        psum_ref[...] = jnp.zeros_like(psum_ref)
        psq_ref[...] = jnp.zeros_like(psq_ref)

    x = x_ref[...].astype(jnp.float32)
    if needs_mask:
        j = pl.program_id(0)
        row0 = (j * steps_per_split + k) * tile
        rows = row0 + jax.lax.broadcasted_iota(jnp.int32, (tile, 1), 0)
        x = jnp.where(rows < n_rows, x, 0.0)

    xr = x.reshape(tile // 8, 8, x.shape[-1])
    psum_ref[...] += jnp.sum(xr, axis=0)
    psq_ref[...] += jnp.sum(xr * xr, axis=0)


def _apply_kernel(x_ref, psum_ref, psq_ref, gamma_ref, beta_ref, o_ref, *,
                  inv_n, eps):
    mean = jnp.sum(psum_ref[...], axis=0, keepdims=True) * inv_n
    msq = jnp.sum(psq_ref[...], axis=0, keepdims=True) * inv_n
    var = jnp.maximum(msq - mean * mean, 0.0)
    scale = gamma_ref[...] * jax.lax.rsqrt(var + eps)
    shift = beta_ref[...] - mean * scale
    o_ref[...] = (x_ref[...].astype(jnp.float32) * scale
                  + shift).astype(o_ref.dtype)


def _round_up(a, m):
    return ((a + m - 1) // m) * m


def _fused_kernel(x_ref, gamma_ref, beta_ref, o_ref, xs_ref, psum_ref,
                  psq_ref, ss_ref, *, s, tile, hc, inv_n, eps):
    k = pl.program_id(1)

    @pl.when(k == 0)
    def _():
        psum_ref[...] = jnp.zeros_like(psum_ref)
        psq_ref[...] = jnp.zeros_like(psq_ref)

    @pl.when(k < s)
    def _():
        x = x_ref[...]
        xr = x.reshape(tile // 8, 8, hc)
        psum_ref[...] += jnp.sum(xr, axis=0)
        psq_ref[...] += jnp.sum(xr * xr, axis=0)

        @pl.when(k < s - 1)
        def _():
            kk = jnp.minimum(k, s - 2)
            xs_ref[pl.ds(kk * tile, tile), :] = x

        @pl.when(k == s - 1)
        def _():
            mean = jnp.sum(psum_ref[...], axis=0, keepdims=True) * inv_n
            msq = jnp.sum(psq_ref[...], axis=0, keepdims=True) * inv_n
            var = jnp.maximum(msq - mean * mean, 0.0)
            scale = gamma_ref[...] * jax.lax.rsqrt(var + eps)
            shift = beta_ref[...] - mean * scale
            ss_ref[0:1, :] = scale
            ss_ref[1:2, :] = shift
            o_ref[...] = x * scale + shift

    @pl.when(k >= s)
    def _():
        r = jnp.maximum(k - s, 0)
        scale = ss_ref[0:1, :]
        shift = ss_ref[1:2, :]
        xb = xs_ref[pl.ds(r * tile, tile), :]
        o_ref[...] = xb * scale + shift


def _fused_single_read(x2, gamma2, beta2, *, tile, eps):
    R, H = x2.shape
    hc = H // 2
    s = R // tile
    inv_n = 1.0 / R

    x_spec = pl.BlockSpec((tile, hc), lambda j, k: (jnp.minimum(k, s - 1), j))
    o_spec = pl.BlockSpec((tile, hc),
                          lambda j, k: (jnp.where(k >= s, k - s, s - 1), j))
    chan_spec = pl.BlockSpec((1, hc), lambda j, k: (0, j))
    return pl.pallas_call(
        functools.partial(_fused_kernel, s=s, tile=tile, hc=hc,
                          inv_n=inv_n, eps=eps),
        grid=(2, 2 * s - 1),
        in_specs=[x_spec, chan_spec, chan_spec],
        out_specs=o_spec,
        out_shape=jax.ShapeDtypeStruct((R, H), x2.dtype),
        scratch_shapes=[
            pltpu.VMEM(((s - 1) * tile, hc), jnp.float32),
            pltpu.VMEM((8, hc), jnp.float32),
            pltpu.VMEM((8, hc), jnp.float32),
            pltpu.VMEM((2, hc), jnp.float32),
        ],
        compiler_params=pltpu.CompilerParams(
            dimension_semantics=("parallel", "arbitrary"),
            vmem_limit_bytes=_VMEM_LIMIT),
    )(x2, gamma2, beta2)


def _manual_kernel(x_hbm, gamma_ref, beta_ref, o_hbm, xs_ref, in_sems,
                   out_sems, *, s, tile, hc, nc, inv_n, eps):
    j = pl.program_id(0)
    hw = hc // nc

    for c in range(nc):
        col = j * hc + c * hw
        for i in range(s):
            pltpu.make_async_copy(
                x_hbm.at[pl.ds(i * tile, tile), pl.ds(col, hw)],
                xs_ref.at[c, i], in_sems.at[c, i]).start()

    for c in range(nc):
        col = j * hc + c * hw
        acc_s = jnp.zeros((8, hw), jnp.float32)
        acc_q = jnp.zeros((8, hw), jnp.float32)
        for i in range(s):
            pltpu.make_async_copy(xs_ref.at[c, i], xs_ref.at[c, i],
                                  in_sems.at[c, i]).wait()
            xr = xs_ref[c, i].reshape(tile // 8, 8, hw)
            acc_s = acc_s + jnp.sum(xr, axis=0)
            acc_q = acc_q + jnp.sum(xr * xr, axis=0)
        mean = jnp.sum(acc_s, axis=0, keepdims=True) * inv_n
        msq = jnp.sum(acc_q, axis=0, keepdims=True) * inv_n
        var = jnp.maximum(msq - mean * mean, 0.0)
        scale = gamma_ref[0:1, c * hw:(c + 1) * hw] * jax.lax.rsqrt(var + eps)
        shift = beta_ref[0:1, c * hw:(c + 1) * hw] - mean * scale
        for i in range(s):
            xs_ref[c, i] = xs_ref[c, i] * scale + shift
            pltpu.make_async_copy(
                xs_ref.at[c, i],
                o_hbm.at[pl.ds(i * tile, tile), pl.ds(col, hw)],
                out_sems.at[c, i]).start(priority=1)

    for c in range(nc):
        for i in range(s):
            pltpu.make_async_copy(xs_ref.at[c, i], xs_ref.at[c, i],
                                  out_sems.at[c, i]).wait()


def _manual_single_read(x2, gamma2, beta2, *, tile, nc, eps):
    R, H = x2.shape
    hc = H // 2
    hw = hc // nc
    s = R // tile
    inv_n = 1.0 / R

    chan_spec = pl.BlockSpec((1, hc), lambda j: (0, j))
    return pl.pallas_call(
        functools.partial(_manual_kernel, s=s, tile=tile, hc=hc, nc=nc,
                          inv_n=inv_n, eps=eps),
        grid=(2,),
        in_specs=[pl.BlockSpec(memory_space=pl.ANY), chan_spec, chan_spec],
        out_specs=pl.BlockSpec(memory_space=pl.ANY),
        out_shape=jax.ShapeDtypeStruct((R, H), x2.dtype),
        scratch_shapes=[
            pltpu.VMEM((nc, s, tile, hw), jnp.float32),
            pltpu.SemaphoreType.DMA((nc, s)),
            pltpu.SemaphoreType.DMA((nc, s)),
        ],
        compiler_params=pltpu.CompilerParams(
            dimension_semantics=("parallel",),
            vmem_limit_bytes=_VMEM_LIMIT),
    )(x2, gamma2, beta2)


def kernel(x, gamma, beta, *, eps=1e-5):
    orig_shape = x.shape
    H = orig_shape[-1]
    x2 = x.reshape(-1, H)
    R = x2.shape[0]
    itemsize = jnp.dtype(x.dtype).itemsize

    gamma2 = gamma.reshape(1, H).astype(jnp.float32)
    beta2 = beta.reshape(1, H).astype(jnp.float32)

    if itemsize == 4 and H % 256 == 0:
        hc = H // 2
        ftile = max(8, (8 * 1024 * 1024 // (hc * 4)) // 8 * 8)
        if (R % ftile == 0 and R // ftile >= 2 and hc % 256 == 0
                and R * hc * 4 <= 36 * 1024 * 1024):
            y2 = _manual_single_read(x2.astype(jnp.float32), gamma2, beta2,
                                     tile=ftile, nc=2, eps=eps)
            return y2.reshape(orig_shape).astype(x.dtype)

    align = 8 if itemsize == 4 else (16 if itemsize == 2 else 32)
    target_bytes = 8 * 1024 * 1024
    bytes_per_row = H * itemsize
    tile = max(align, (target_bytes // bytes_per_row) // align * align)
    tile = min(tile, _round_up(R, align))

    steps_total = pl.cdiv(R, tile)
    splits = 2 if steps_total >= 2 else 1
    steps_per_split = pl.cdiv(steps_total, splits)
    covered = splits * steps_per_split
    needs_mask = covered * tile != R
    needs_clamp = covered > steps_total
    last_block = steps_total - 1

    if needs_clamp:
        def x_stats_map(j, k):
            return (jnp.minimum(j * steps_per_split + k, last_block), 0)
    else:
        def x_stats_map(j, k):
            return (j * steps_per_split + k, 0)

    psum, psq = pl.pallas_call(
        functools.partial(_stats_kernel, n_rows=R, tile=tile,
                          steps_per_split=steps_per_split,
                          needs_mask=needs_mask),
        grid=(splits, steps_per_split),
        in_specs=[pl.BlockSpec((tile, H), x_stats_map)],
        out_specs=(pl.BlockSpec((None, 8, H), lambda j, k: (j, 0, 0)),
                   pl.BlockSpec((None, 8, H), lambda j, k: (j, 0, 0))),
        out_shape=(jax.ShapeDtypeStruct((splits, 8, H), jnp.float32),
                   jax.ShapeDtypeStruct((splits, 8, H), jnp.float32)),
        compiler_params=pltpu.CompilerParams(
            dimension_semantics=("parallel", "arbitrary"),
            vmem_limit_bytes=_VMEM_LIMIT),
    )(x2)

    psum2 = psum.reshape(splits * 8, H)
    psq2 = psq.reshape(splits * 8, H)
    atile = max(align, (8 * 1024 * 1024 // bytes_per_row) // align * align)
    atile = min(atile, _round_up(R, align))
    asteps = pl.cdiv(R, atile)
    row_spec = pl.BlockSpec((atile, H), lambda i: (i, 0))
    part_spec = pl.BlockSpec((splits * 8, H), lambda i: (0, 0))
    chan_spec = pl.BlockSpec((1, H), lambda i: (0, 0))
    y2 = pl.pallas_call(
        functools.partial(_apply_kernel, inv_n=1.0 / R, eps=eps),
        grid=(asteps,),
        in_specs=[row_spec, part_spec, part_spec, chan_spec, chan_spec],
        out_specs=row_spec,
        out_shape=jax.ShapeDtypeStruct((R, H), x.dtype),
        compiler_params=pltpu.CompilerParams(
            dimension_semantics=("parallel",),
            vmem_limit_bytes=_VMEM_LIMIT),
    )(x2, psum2, psq2, gamma2, beta2)

    return y2.reshape(orig_shape)
